# baseline (speedup 1.0000x reference)
import functools

import jax
import jax.numpy as jnp
from jax import lax
from jax.experimental import pallas as pl
from jax.experimental.pallas import tpu as pltpu
from jax.experimental.pallas import tpu_sc as plsc

B, A, O, K, D = 1024, 8, 64, 32, 128
N = B * A * O
NC, NS = 2, 16
NW = NC * NS
PER_W = N // NW
CROWS = 128
NCHK = PER_W // CROWS
NBUF = 6
GROUPS = CROWS // 16


def _table_body(w_ref, b_ref, t_ref):
    t_ref[...] = w_ref[...] + b_ref[...]


def _fused_table(W, b):
    return pl.pallas_call(
        _table_body,
        out_shape=jax.ShapeDtypeStruct((K, D), jnp.float32),
    )(W, b.reshape(1, D))


_mesh = plsc.VectorSubcoreMesh(core_axis_name="c", subcore_axis_name="s")


@functools.partial(
    pl.kernel,
    mesh=_mesh,
    out_type=jax.ShapeDtypeStruct((N, D), jnp.float32),
    compiler_params=pltpu.CompilerParams(needs_layout_passes=False),
    scratch_types=[
        pltpu.VMEM((NBUF, CROWS), jnp.int32),
        pltpu.VMEM((K, D), jnp.float32),
        pltpu.VMEM((NBUF, CROWS, D), jnp.float32),
        pltpu.SemaphoreType.DMA((NBUF,)),
        pltpu.SemaphoreType.DMA((NBUF,)),
    ],
)
def _sc_lookup(table_hbm, idx_hbm, out_hbm, idx_vv, t_v, obuf, sem_s, sem_i):
    wid = lax.axis_index("s") * NC + lax.axis_index("c")
    base = wid * PER_W
    pltpu.sync_copy(table_hbm, t_v)

    def idx_dma(ci, b):
        return pltpu.make_async_copy(
            idx_hbm.at[wid, jnp.minimum(ci, NCHK - 1)], idx_vv.at[b],
            sem_i.at[b])

    def build(b):
        @plsc.parallel_loop(0, CROWS // 16, unroll=2)
        def group(gr):
            x_vec = idx_vv[b, pl.ds(gr * 16, 16)]
            for u in range(16):
                xi = x_vec[u]
                r = gr * 16 + u
                for jj in range(D // 16):
                    obuf[b, r, pl.ds(16 * jj, 16)] = t_v[xi, pl.ds(16 * jj, 16)]

    def scatter(ci, b):
        return pltpu.make_async_copy(
            obuf.at[b], out_hbm.at[pl.ds(base + ci * CROWS, CROWS)],
            sem_s.at[b])

    idx_dma(0, 0).start()

    def prologue(ci, carry):
        idx_dma(ci, ci).wait()
        idx_dma(ci + 1, lax.rem(ci + 1, NBUF)).start()
        build(ci)
        scatter(ci, ci).start()
        return carry

    lax.fori_loop(0, NBUF, prologue, 0)

    def body(ci, carry):
        bb = lax.rem(ci, NBUF)
        scatter(ci - NBUF, bb).wait()
        idx_dma(ci, bb).wait()
        idx_dma(ci + 1, lax.rem(ci + 1, NBUF)).start()
        build(bb)
        scatter(ci, bb).start()
        return carry

    lax.fori_loop(NBUF, NCHK, body, 0)

    idx_dma(NCHK - 1, NCHK % NBUF).wait()

    def drain(ci, carry):
        scatter(ci, lax.rem(ci, NBUF)).wait()
        return carry

    lax.fori_loop(NCHK - NBUF, NCHK, drain, 0)


def kernel(x, W, b):
    table = _fused_table(W, b)
    idx = x.astype(jnp.int32).reshape(NW, NCHK, CROWS)
    y = _sc_lookup(table, idx)
    return y.reshape(B, A, O, D)


# speedup vs baseline: 1.0026x; 1.0026x over previous
import functools

import jax
import jax.numpy as jnp
from jax import lax
from jax.experimental import pallas as pl
from jax.experimental.pallas import tpu as pltpu
from jax.experimental.pallas import tpu_sc as plsc

B, A, O, K, D = 1024, 8, 64, 32, 128
N = B * A * O
NC, NS = 2, 16
NW = NC * NS
PER_W = N // NW
CROWS = 128
NCHK = PER_W // CROWS
NBUF = 4
GROUPS = CROWS // 16


def _table_body(w_ref, b_ref, t_ref):
    t_ref[...] = w_ref[...] + b_ref[...]


def _fused_table(W, b):
    return pl.pallas_call(
        _table_body,
        out_shape=jax.ShapeDtypeStruct((K, D), jnp.float32),
    )(W, b.reshape(1, D))


_mesh = plsc.VectorSubcoreMesh(core_axis_name="c", subcore_axis_name="s")


@functools.partial(
    pl.kernel,
    mesh=_mesh,
    out_type=jax.ShapeDtypeStruct((N, D), jnp.float32),
    compiler_params=pltpu.CompilerParams(needs_layout_passes=False),
    scratch_types=[
        pltpu.VMEM((NBUF, CROWS), jnp.int32),
        pltpu.VMEM((K, D), jnp.float32),
        pltpu.VMEM((NBUF, CROWS, D), jnp.float32),
        pltpu.SemaphoreType.DMA((NBUF,)),
        pltpu.SemaphoreType.DMA((NBUF,)),
    ],
)
def _sc_lookup(table_hbm, idx_hbm, out_hbm, idx_vv, t_v, obuf, sem_s, sem_i):
    wid = lax.axis_index("s") * NC + lax.axis_index("c")
    base = wid * PER_W
    pltpu.sync_copy(table_hbm, t_v)

    def idx_dma(ci, b):
        return pltpu.make_async_copy(
            idx_hbm.at[wid, jnp.minimum(ci, NCHK - 1)], idx_vv.at[b],
            sem_i.at[b])

    def build(b):
        @plsc.parallel_loop(0, CROWS // 16, unroll=2)
        def group(gr):
            x_vec = idx_vv[b, pl.ds(gr * 16, 16)]
            for u in range(16):
                xi = x_vec[u]
                r = gr * 16 + u
                for jj in range(D // 16):
                    obuf[b, r, pl.ds(16 * jj, 16)] = t_v[xi, pl.ds(16 * jj, 16)]

    def scatter(ci, b):
        return pltpu.make_async_copy(
            obuf.at[b], out_hbm.at[pl.ds(base + ci * CROWS, CROWS)],
            sem_s.at[b])

    idx_dma(0, 0).start()

    def prologue(ci, carry):
        idx_dma(ci, ci).wait()
        idx_dma(ci + 1, lax.rem(ci + 1, NBUF)).start()
        build(ci)
        scatter(ci, ci).start()
        return carry

    lax.fori_loop(0, NBUF, prologue, 0)

    def body(ci, carry):
        bb = lax.rem(ci, NBUF)
        scatter(ci - NBUF, bb).wait()
        idx_dma(ci, bb).wait()
        idx_dma(ci + 1, lax.rem(ci + 1, NBUF)).start()
        build(bb)
        scatter(ci, bb).start()
        return carry

    lax.fori_loop(NBUF, NCHK, body, 0)

    idx_dma(NCHK - 1, NCHK % NBUF).wait()

    def drain(ci, carry):
        scatter(ci, lax.rem(ci, NBUF)).wait()
        return carry

    lax.fori_loop(NCHK - NBUF, NCHK, drain, 0)


def kernel(x, W, b):
    table = _fused_table(W, b)
    idx = x.astype(jnp.int32).reshape(NW, NCHK, CROWS)
    y = _sc_lookup(table, idx)
    return y.reshape(B, A, O, D)
